# baseline (speedup 1.0000x reference)
import functools
import math

import jax
import jax.numpy as jnp
from jax import lax
from jax.experimental import pallas as pl
from jax.experimental.pallas import tpu as pltpu
from jax.experimental.pallas import tpu_sc as plsc

D_MODEL = 64
SCALE = math.sqrt(D_MODEL)
NUM_WORKERS = 32
CHUNK = 1024
LANES = 16


@functools.partial(
    pl.kernel,
    mesh=plsc.VectorSubcoreMesh(core_axis_name="c", subcore_axis_name="s"),
    out_type=jax.ShapeDtypeStruct((819200, D_MODEL), jnp.float32),
    scratch_types=[
        pltpu.VMEM((819200 // NUM_WORKERS,), jnp.int32),
        pltpu.VMEM((CHUNK, D_MODEL), jnp.float32),
        pltpu.SemaphoreType.DMA,
    ],
    compiler_params=pltpu.CompilerParams(use_tc_tiling_on_sc=False),
)
def _emb_lookup(x_hbm, lut_hbm, out_hbm, idx_v, rows_v, sem):
    b_per_w = 819200 // NUM_WORKERS
    wid = lax.axis_index("s") * 2 + lax.axis_index("c")
    base = wid * b_per_w

    pltpu.sync_copy(x_hbm.at[pl.ds(base, b_per_w)], idx_v)

    def chunk_body(i, _):
        off = i * CHUNK
        pltpu.async_copy(
            lut_hbm.at[idx_v.at[pl.ds(off, CHUNK)]], rows_v, sem
        ).wait()

        def scale_row(r, _):
            for j in range(D_MODEL // LANES):
                sl = pl.ds(j * LANES, LANES)
                rows_v[r, sl] = rows_v[r, sl] * SCALE
            return 0

        lax.fori_loop(0, CHUNK, scale_row, 0, unroll=4)

        pltpu.sync_copy(rows_v, out_hbm.at[pl.ds(base + off, CHUNK)])
        return 0

    lax.fori_loop(0, b_per_w // CHUNK, chunk_body, 0)


def kernel(x, lut):
    b, s = x.shape
    out = _emb_lookup(x.reshape(-1).astype(jnp.int32), lut)
    return out.reshape(b, s, D_MODEL)


# speedup vs baseline: 1.0537x; 1.0537x over previous
import functools
import math

import jax
import jax.numpy as jnp
from jax import lax
from jax.experimental import pallas as pl
from jax.experimental.pallas import tpu as pltpu
from jax.experimental.pallas import tpu_sc as plsc

D_MODEL = 64
SCALE = math.sqrt(D_MODEL)
NUM_WORKERS = 32
B_TOTAL = 819200
B_PER_W = B_TOTAL // NUM_WORKERS
CHUNK = 512
N_CHUNKS = B_PER_W // CHUNK
LANES = 16


@functools.partial(
    pl.kernel,
    mesh=plsc.VectorSubcoreMesh(core_axis_name="c", subcore_axis_name="s"),
    out_type=jax.ShapeDtypeStruct((B_TOTAL, D_MODEL), jnp.float32),
    scratch_types=[
        pltpu.VMEM((B_PER_W,), jnp.int32),
        pltpu.VMEM((CHUNK, D_MODEL), jnp.float32),
        pltpu.VMEM((CHUNK, D_MODEL), jnp.float32),
        pltpu.SemaphoreType.DMA,
        pltpu.SemaphoreType.DMA,
        pltpu.SemaphoreType.DMA,
        pltpu.SemaphoreType.DMA,
    ],
    compiler_params=pltpu.CompilerParams(use_tc_tiling_on_sc=False),
)
def _emb_lookup(x_hbm, lut_hbm, out_hbm, idx_v, buf0, buf1, g0, g1, s0, s1):
    wid = lax.axis_index("s") * 2 + lax.axis_index("c")
    base = wid * B_PER_W

    pltpu.sync_copy(x_hbm.at[pl.ds(base, B_PER_W)], idx_v)

    def start_gather(c, buf, gsem):
        pltpu.make_async_copy(
            lut_hbm.at[idx_v.at[pl.ds(c * CHUNK, CHUNK)]], buf, gsem
        ).start()

    def wait_gather(buf, gsem):
        pltpu.make_async_copy(
            lut_hbm.at[idx_v.at[pl.ds(0, CHUNK)]], buf, gsem
        ).wait()

    def start_scatter(c, buf, ssem):
        pltpu.make_async_copy(
            buf, out_hbm.at[pl.ds(base + c * CHUNK, CHUNK)], ssem
        ).start()

    def wait_scatter(buf, ssem):
        pltpu.make_async_copy(
            buf, out_hbm.at[pl.ds(base, CHUNK)], ssem
        ).wait()

    def scale(buf):
        def scale_row(r, _):
            for j in range(D_MODEL // LANES):
                sl = pl.ds(j * LANES, LANES)
                buf[r, sl] = buf[r, sl] * SCALE
            return 0

        lax.fori_loop(0, CHUNK, scale_row, 0, unroll=8)

    start_gather(0, buf0, g0)
    wait_gather(buf0, g0)
    start_gather(1, buf1, g1)
    scale(buf0)
    start_scatter(0, buf0, s0)

    def pair_body(kk, _):
        c1 = 2 * kk + 1
        wait_gather(buf1, g1)
        wait_scatter(buf0, s0)
        start_gather(c1 + 1, buf0, g0)
        scale(buf1)
        start_scatter(c1, buf1, s1)

        c2 = c1 + 1
        wait_gather(buf0, g0)
        wait_scatter(buf1, s1)
        start_gather(c2 + 1, buf1, g1)
        scale(buf0)
        start_scatter(c2, buf0, s0)
        return 0

    lax.fori_loop(0, (N_CHUNKS - 2) // 2, pair_body, 0)

    wait_gather(buf1, g1)
    scale(buf1)
    start_scatter(N_CHUNKS - 1, buf1, s1)
    wait_scatter(buf0, s0)
    wait_scatter(buf1, s1)


def kernel(x, lut):
    b, s = x.shape
    out = _emb_lookup(x.reshape(-1).astype(jnp.int32), lut)
    return out.reshape(b, s, D_MODEL)
